# Initial kernel scaffold; baseline (speedup 1.0000x reference)
import jax
import jax.numpy as jnp
from jax.experimental import pallas as pl


def kernel(x, edge_index, W, att_src, att_dst, bias, bn_weight, bn_bias):
    raise NotImplementedError("write your pallas kernel here")


# speedup vs baseline: 38.9066x; 38.9066x over previous
import functools

import jax
import jax.numpy as jnp
from jax import lax
from jax.experimental import pallas as pl
from jax.experimental.pallas import tpu as pltpu
from jax.experimental.pallas import tpu_sc as plsc

N = 10000
E = 320000
IN = 128
H = 16
C = 8
OUT = H * C

NC = 2
NS = 16
NW = NC * NS
EPT = E // NW
K = 80
NCHUNK = EPT // K
NP = 10240
RPT = NP // NS


def _proj_body(x_ref, wp_ref, ats_ref, atd_ref, ht_ref, as_ref, ad_ref, mh_ref):
    ht = jnp.dot(x_ref[...], wp_ref[...], preferred_element_type=jnp.float32)
    ht_ref[...] = ht
    a_s = jnp.dot(ht, ats_ref[...], preferred_element_type=jnp.float32)
    a_d = jnp.dot(ht, atd_ref[...], preferred_element_type=jnp.float32)
    as_ref[...] = a_s
    ad_ref[...] = a_d
    m = jnp.max(a_s, axis=0, keepdims=True) + jnp.max(a_d, axis=0, keepdims=True)
    mh_ref[...] = jnp.where(m > 0.0, m, 0.2 * m)


def _sc_body(src_hbm, dst_hbm, asrc_hbm, adst_hbm, ht_hbm, mh_hbm, z3_hbm,
             z2_hbm, acc_out, den_out, acc_sh, den_sh, sidx, didx, asb, adb,
             htb, msgb, ewb, mhv, sem1, sem2, sem3):
    cid = lax.axis_index("c")
    sid = lax.axis_index("s")
    wid = cid * NS + sid
    r0 = sid * RPT
    pltpu.sync_copy(z3_hbm.at[pl.ds(r0, RPT)], acc_sh.at[pl.ds(r0, RPT)])
    pltpu.sync_copy(z2_hbm.at[pl.ds(r0, RPT)], den_sh.at[pl.ds(r0, RPT)])
    pltpu.sync_copy(mh_hbm, mhv)
    plsc.subcore_barrier()
    m = mhv[...]
    ebase = wid * EPT

    def chunk(i, carry):
        base = ebase + i * K
        pltpu.sync_copy(src_hbm.at[pl.ds(base, K)], sidx)
        pltpu.sync_copy(dst_hbm.at[pl.ds(base, K)], didx)
        g1 = pltpu.async_copy(asrc_hbm.at[sidx], asb, sem1)
        g2 = pltpu.async_copy(adst_hbm.at[didx], adb, sem2)
        g3 = pltpu.async_copy(ht_hbm.at[sidx], htb, sem3)
        g1.wait()
        g2.wait()
        g3.wait()

        def edge(j, c2):
            s = asb[j] + adb[j]
            s = jnp.where(s > 0.0, s, 0.2 * s)
            e = jnp.exp(s - m)
            ewb[j] = e
            for c in range(C):
                msgb[j, c] = htb[j, c] * e
            return c2

        lax.fori_loop(0, K, edge, 0)
        pltpu.sync_copy(ewb, den_sh.at[didx], add=True)
        pltpu.sync_copy(msgb, acc_sh.at[didx], add=True)
        return carry

    lax.fori_loop(0, NCHUNK, chunk, 0)
    plsc.subcore_barrier()
    pltpu.sync_copy(acc_sh.at[pl.ds(r0, RPT)], acc_out.at[cid, pl.ds(r0, RPT)])
    pltpu.sync_copy(den_sh.at[pl.ds(r0, RPT)], den_out.at[cid, pl.ds(r0, RPT)])


def _epi_body(acc_ref, den_ref, p_ref, t_ref, b_ref, g_ref, bb_ref, out_ref):
    a = acc_ref[0][:N] + acc_ref[1][:N]
    d = den_ref[0][:N] + den_ref[1][:N] + 1e-16
    dt = jnp.dot(d, t_ref[...], preferred_element_type=jnp.float32)
    agg = jnp.dot(a / dt, p_ref[...], preferred_element_type=jnp.float32)
    scale = g_ref[...] * (1.0 / jnp.sqrt(1.0 + 1e-5))
    o = (agg + b_ref[...]) * scale + bb_ref[...]
    out_ref[...] = jnp.maximum(o, 0.0)


def kernel(x, edge_index, W, att_src, att_dst, bias, bn_weight, bn_bias):
    f32 = jnp.float32
    t = jnp.arange(OUT)
    t_to_o = (t % H) * C + (t // H)
    Wp = W[:, t_to_o]
    ats = jnp.zeros((OUT, H), f32).at[t, t % H].set(att_src[t % H, t // H])
    atd = jnp.zeros((OUT, H), f32).at[t, t % H].set(att_dst[t % H, t // H])
    P = jnp.zeros((OUT, OUT), f32).at[t, t_to_o].set(1.0)
    T = jnp.zeros((H, OUT), f32).at[t % H, t].set(1.0)
    src = edge_index[0].astype(jnp.int32)
    dst = edge_index[1].astype(jnp.int32)

    ht, asrc, adst, mh = pl.pallas_call(
        _proj_body,
        out_shape=[
            jax.ShapeDtypeStruct((N, OUT), f32),
            jax.ShapeDtypeStruct((N, H), f32),
            jax.ShapeDtypeStruct((N, H), f32),
            jax.ShapeDtypeStruct((1, H), f32),
        ],
    )(x, Wp, ats, atd)

    ht3 = ht.reshape(N, C, H)
    z3 = jnp.zeros((NP, C, H), f32)
    z2 = jnp.zeros((NP, H), f32)

    sc_edge = pl.kernel(
        _sc_body,
        out_type=[
            jax.ShapeDtypeStruct((NC, NP, C, H), f32),
            jax.ShapeDtypeStruct((NC, NP, H), f32),
        ],
        mesh=plsc.VectorSubcoreMesh(core_axis_name="c", subcore_axis_name="s"),
        compiler_params=pltpu.CompilerParams(use_tc_tiling_on_sc=False),
        scratch_types=[
            pltpu.VMEM_SHARED((NP, C, H), f32),
            pltpu.VMEM_SHARED((NP, H), f32),
            pltpu.VMEM((K,), jnp.int32),
            pltpu.VMEM((K,), jnp.int32),
            pltpu.VMEM((K, H), f32),
            pltpu.VMEM((K, H), f32),
            pltpu.VMEM((K, C, H), f32),
            pltpu.VMEM((K, C, H), f32),
            pltpu.VMEM((K, H), f32),
            pltpu.VMEM((H,), f32),
            pltpu.SemaphoreType.DMA,
            pltpu.SemaphoreType.DMA,
            pltpu.SemaphoreType.DMA,
        ],
    )
    acc, den = sc_edge(src, dst, asrc, adst, ht3, mh.reshape(H), z3, z2)

    out = pl.pallas_call(
        _epi_body,
        out_shape=jax.ShapeDtypeStruct((N, OUT), f32),
    )(acc.reshape(NC, NP, OUT), den, P, T, bias.reshape(1, OUT),
      bn_weight.reshape(1, OUT), bn_bias.reshape(1, OUT))
    return out


# speedup vs baseline: 86.7383x; 2.2294x over previous
import functools

import jax
import jax.numpy as jnp
from jax import lax
from jax.experimental import pallas as pl
from jax.experimental.pallas import tpu as pltpu
from jax.experimental.pallas import tpu_sc as plsc

N = 10000
E = 320000
IN = 128
H = 16
C = 8
OUT = H * C
R = C + 1

NC = 2
NS = 16
NW = NC * NS
EPT = E // NW
K = 40
NCHUNK = EPT // K
NP = 10240
RPT = NP // NS


def _proj_body(x_ref, wp_ref, ats_ref, atd_ref, hta_ref, ad_ref, mh_ref):
    ht = jnp.dot(x_ref[...], wp_ref[...], preferred_element_type=jnp.float32)
    a_s = jnp.dot(ht, ats_ref[...], preferred_element_type=jnp.float32)
    a_d = jnp.dot(ht, atd_ref[...], preferred_element_type=jnp.float32)
    hta_ref[...] = jnp.concatenate([ht, a_s], axis=1)
    ad_ref[...] = a_d
    m = jnp.max(a_s, axis=0, keepdims=True) + jnp.max(a_d, axis=0, keepdims=True)
    mh_ref[...] = jnp.where(m > 0.0, m, 0.2 * m)


def _sc_body(src_hbm, dst_hbm, hta_hbm, adst_hbm, mh_hbm, z3_hbm, acc_out,
             acc_sh, sidx_a, sidx_b, didx, htab_a, htab_b, adb_a, adb_b,
             msgw_a, msgw_b, mhv, semg_a, semg_b, sems_a, sems_b, semi_a,
             semi_b):
    cid = lax.axis_index("c")
    sid = lax.axis_index("s")
    wid = cid * NS + sid
    r0 = sid * RPT
    cbase = wid * NCHUNK
    pltpu.sync_copy(z3_hbm.at[pl.ds(r0, RPT)], acc_sh.at[pl.ds(r0, RPT)])
    pltpu.sync_copy(mh_hbm, mhv)
    pltpu.sync_copy(dst_hbm.at[wid], didx)
    pltpu.sync_copy(src_hbm.at[cbase], sidx_a)
    pltpu.sync_copy(src_hbm.at[cbase + 1], sidx_b)
    plsc.subcore_barrier()
    m = mhv[...]

    def fire(c, sidx, htab, adb, semg):
        pltpu.async_copy(hta_hbm.at[sidx], htab, semg)
        pltpu.async_copy(adst_hbm.at[didx.at[c]], adb, semg)

    def drain_gather(sidx, htab, adb, semg):
        pltpu.make_async_copy(hta_hbm.at[sidx], htab, semg).wait()
        pltpu.make_async_copy(adst_hbm.at[didx.at[0]], adb, semg).wait()

    def compute(htab, adb, msgw):
        @plsc.parallel_loop(0, K, unroll=4)
        def _(j):
            s = htab[j, C] + adb[j]
            s = jnp.where(s > 0.0, s, 0.2 * s)
            e = jnp.exp(s - m)
            msgw[j, C] = e
            for c in range(C):
                msgw[j, c] = htab[j, c] * e

    def wait_scatter(msgw, sems):
        pltpu.make_async_copy(msgw, acc_sh.at[didx.at[0]], sems).wait()

    def phase(i2, ca, sidx, htab, adb, msgw, semg, sems, semi):
        drain_gather(sidx, htab, adb, semg)

        @pl.when(i2 > 0)
        def _():
            wait_scatter(msgw, sems)

        @pl.when(ca + 2 < NCHUNK)
        def _():
            pltpu.async_copy(src_hbm.at[cbase + ca + 2], sidx, semi)

        compute(htab, adb, msgw)
        pltpu.async_copy(msgw, acc_sh.at[didx.at[ca]], sems, add=True)

        @pl.when(ca + 2 < NCHUNK)
        def _():
            pltpu.make_async_copy(src_hbm.at[cbase], sidx, semi).wait()
            fire(ca + 2, sidx, htab, adb, semg)

    fire(0, sidx_a, htab_a, adb_a, semg_a)
    fire(1, sidx_b, htab_b, adb_b, semg_b)

    def step2(i2, carry):
        ca = 2 * i2
        phase(i2, ca, sidx_a, htab_a, adb_a, msgw_a, semg_a, sems_a, semi_a)
        phase(i2, ca + 1, sidx_b, htab_b, adb_b, msgw_b, semg_b, sems_b,
              semi_b)
        return carry

    lax.fori_loop(0, NCHUNK // 2, step2, 0)
    wait_scatter(msgw_a, sems_a)
    wait_scatter(msgw_b, sems_b)
    plsc.subcore_barrier()
    pltpu.sync_copy(acc_sh.at[pl.ds(r0, RPT)], acc_out.at[cid, pl.ds(r0, RPT)])


def _epi_body(acc_ref, p_ref, t_ref, b_ref, g_ref, bb_ref, out_ref):
    a = acc_ref[0][:N] + acc_ref[1][:N]
    msg = a[:, :OUT]
    d = a[:, OUT:OUT + H] + 1e-16
    dt = jnp.dot(d, t_ref[...], preferred_element_type=jnp.float32)
    agg = jnp.dot(msg / dt, p_ref[...], preferred_element_type=jnp.float32)
    scale = g_ref[...] * (1.0 / jnp.sqrt(1.0 + 1e-5))
    o = (agg + b_ref[...]) * scale + bb_ref[...]
    out_ref[...] = jnp.maximum(o, 0.0)


def kernel(x, edge_index, W, att_src, att_dst, bias, bn_weight, bn_bias):
    f32 = jnp.float32
    t = jnp.arange(OUT)
    t_to_o = (t % H) * C + (t // H)
    Wp = W[:, t_to_o]
    ats = jnp.zeros((OUT, H), f32).at[t, t % H].set(att_src[t % H, t // H])
    atd = jnp.zeros((OUT, H), f32).at[t, t % H].set(att_dst[t % H, t // H])
    P = jnp.zeros((OUT, OUT), f32).at[t, t_to_o].set(1.0)
    T = jnp.zeros((H, OUT), f32).at[t % H, t].set(1.0)
    src = edge_index[0].astype(jnp.int32)
    dst = edge_index[1].astype(jnp.int32)

    hta, adst, mh = pl.pallas_call(
        _proj_body,
        out_shape=[
            jax.ShapeDtypeStruct((N, R * H), f32),
            jax.ShapeDtypeStruct((N, H), f32),
            jax.ShapeDtypeStruct((1, H), f32),
        ],
    )(x, Wp, ats, atd)

    sc_edge = pl.kernel(
        _sc_body,
        out_type=jax.ShapeDtypeStruct((NC, NP, R, H), f32),
        mesh=plsc.VectorSubcoreMesh(core_axis_name="c", subcore_axis_name="s"),
        compiler_params=pltpu.CompilerParams(use_tc_tiling_on_sc=False),
        scratch_types=[
            pltpu.VMEM_SHARED((NP, R, H), f32),
            pltpu.VMEM((K,), jnp.int32),
            pltpu.VMEM((K,), jnp.int32),
            pltpu.VMEM((NCHUNK, K), jnp.int32),
            pltpu.VMEM((K, R, H), f32),
            pltpu.VMEM((K, R, H), f32),
            pltpu.VMEM((K, H), f32),
            pltpu.VMEM((K, H), f32),
            pltpu.VMEM((K, R, H), f32),
            pltpu.VMEM((K, R, H), f32),
            pltpu.VMEM((H,), f32),
            pltpu.SemaphoreType.DMA,
            pltpu.SemaphoreType.DMA,
            pltpu.SemaphoreType.DMA,
            pltpu.SemaphoreType.DMA,
            pltpu.SemaphoreType.DMA,
            pltpu.SemaphoreType.DMA,
        ],
    )
    acc = sc_edge(src.reshape(NW * NCHUNK, K), dst.reshape(NW, NCHUNK, K),
                  hta.reshape(N, R, H), adst, mh.reshape(H),
                  jnp.zeros((NP, R, H), f32))

    out = pl.pallas_call(
        _epi_body,
        out_shape=jax.ShapeDtypeStruct((N, OUT), f32),
    )(acc.reshape(NC, NP, R * H), P, T, bias.reshape(1, OUT),
      bn_weight.reshape(1, OUT), bn_bias.reshape(1, OUT))
    return out


# speedup vs baseline: 150.5831x; 1.7361x over previous
import functools

import jax
import jax.numpy as jnp
from jax import lax
from jax.experimental import pallas as pl
from jax.experimental.pallas import tpu as pltpu
from jax.experimental.pallas import tpu_sc as plsc

N = 10000
E = 320000
IN = 128
H = 16
C = 8
OUT = H * C

NC = 2
NS = 16
NW = NC * NS
EPT = E // NW
K = 40
NCHUNK = EPT // K
NP = 10240
RPT = NP // NS


def _fold_matrix():
    io = lax.broadcasted_iota(jnp.int32, (OUT, H), 0)
    ik = lax.broadcasted_iota(jnp.int32, (OUT, H), 1)
    return (io % H == ik).astype(jnp.float32)


def _proj_body(x_ref, wp_ref, ats_ref, atd_ref, ht_ref, as_ref, ad_ref,
               mh_ref):
    ht = jnp.dot(x_ref[...], wp_ref[...], preferred_element_type=jnp.float32)
    ht_ref[...] = ht
    m_fold = _fold_matrix()
    a_s = jnp.dot(ht * ats_ref[...], m_fold,
                  preferred_element_type=jnp.float32)
    a_d = jnp.dot(ht * atd_ref[...], m_fold,
                  preferred_element_type=jnp.float32)
    as_ref[...] = a_s
    ad_ref[...] = a_d
    m = jnp.max(a_s, axis=0, keepdims=True) + jnp.max(a_d, axis=0,
                                                      keepdims=True)
    mh_ref[...] = jnp.where(m > 0.0, m, 0.2 * m)


def _sc_body(src_hbm, dst_hbm, ht_hbm, asrc_hbm, adst_hbm, mh_hbm, z128_hbm,
             z16_hbm, acc_out, den_out, acc_sh, den_sh, sidx_a, sidx_b, didx,
             htab_a, htab_b, asb_a, asb_b, adb_a, adb_b, msgw_a, msgw_b,
             ewb_a, ewb_b, mhv, semg_a, semg_b, sems_a, sems_b, semi_a,
             semi_b):
    cid = lax.axis_index("c")
    sid = lax.axis_index("s")
    wid = cid * NS + sid
    r0 = sid * RPT
    cbase = wid * NCHUNK
    pltpu.sync_copy(z128_hbm.at[pl.ds(r0, RPT)], acc_sh.at[pl.ds(r0, RPT)])
    pltpu.sync_copy(z16_hbm.at[pl.ds(r0, RPT)], den_sh.at[pl.ds(r0, RPT)])
    pltpu.sync_copy(mh_hbm, mhv)
    pltpu.sync_copy(dst_hbm.at[wid], didx)
    pltpu.sync_copy(src_hbm.at[cbase], sidx_a)
    pltpu.sync_copy(src_hbm.at[cbase + 1], sidx_b)
    plsc.subcore_barrier()
    m = mhv[...]

    def fire(c, sidx, htab, asb, adb, semg):
        pltpu.async_copy(ht_hbm.at[sidx], htab, semg)
        pltpu.async_copy(asrc_hbm.at[sidx], asb, semg)
        pltpu.async_copy(adst_hbm.at[didx.at[c]], adb, semg)

    def drain_gather(sidx, htab, asb, adb, semg):
        pltpu.make_async_copy(ht_hbm.at[sidx], htab, semg).wait()
        pltpu.make_async_copy(asrc_hbm.at[sidx], asb, semg).wait()
        pltpu.make_async_copy(adst_hbm.at[didx.at[0]], adb, semg).wait()

    def compute(htab, asb, adb, msgw, ewb):
        @plsc.parallel_loop(0, K, unroll=4)
        def _(j):
            s = asb[j] + adb[j]
            s = jnp.where(s > 0.0, s, 0.2 * s)
            e = jnp.exp(s - m)
            ewb[j] = e
            for c in range(C):
                msgw[j, pl.ds(c * H, H)] = htab[j, pl.ds(c * H, H)] * e

    def wait_scatter(msgw, ewb, sems):
        pltpu.make_async_copy(msgw, acc_sh.at[didx.at[0]], sems).wait()
        pltpu.make_async_copy(ewb, den_sh.at[didx.at[0]], sems).wait()

    def phase(i2, ca, sidx, htab, asb, adb, msgw, ewb, semg, sems, semi):
        drain_gather(sidx, htab, asb, adb, semg)

        @pl.when(i2 > 0)
        def _():
            wait_scatter(msgw, ewb, sems)

        @pl.when(ca + 2 < NCHUNK)
        def _():
            pltpu.async_copy(src_hbm.at[cbase + ca + 2], sidx, semi)

        compute(htab, asb, adb, msgw, ewb)
        pltpu.async_copy(msgw, acc_sh.at[didx.at[ca]], sems, add=True)
        pltpu.async_copy(ewb, den_sh.at[didx.at[ca]], sems, add=True)

        @pl.when(ca + 2 < NCHUNK)
        def _():
            pltpu.make_async_copy(src_hbm.at[cbase], sidx, semi).wait()
            fire(ca + 2, sidx, htab, asb, adb, semg)

    fire(0, sidx_a, htab_a, asb_a, adb_a, semg_a)
    fire(1, sidx_b, htab_b, asb_b, adb_b, semg_b)

    def step2(i2, carry):
        ca = 2 * i2
        phase(i2, ca, sidx_a, htab_a, asb_a, adb_a, msgw_a, ewb_a, semg_a,
              sems_a, semi_a)
        phase(i2, ca + 1, sidx_b, htab_b, asb_b, adb_b, msgw_b, ewb_b,
              semg_b, sems_b, semi_b)
        return carry

    lax.fori_loop(0, NCHUNK // 2, step2, 0)
    wait_scatter(msgw_a, ewb_a, sems_a)
    wait_scatter(msgw_b, ewb_b, sems_b)
    plsc.subcore_barrier()
    pltpu.sync_copy(acc_sh.at[pl.ds(r0, RPT)], acc_out.at[cid, pl.ds(r0, RPT)])
    pltpu.sync_copy(den_sh.at[pl.ds(r0, RPT)], den_out.at[cid, pl.ds(r0, RPT)])


def _epi_body(acc_ref, den_ref, b_ref, g_ref, bb_ref, out_ref):
    a = acc_ref[0][:N] + acc_ref[1][:N]
    d = den_ref[0][:N] + den_ref[1][:N] + 1e-16
    ik = lax.broadcasted_iota(jnp.int32, (H, OUT), 0)
    it = lax.broadcasted_iota(jnp.int32, (H, OUT), 1)
    tmat = (it % H == ik).astype(jnp.float32)
    dt = jnp.dot(d, tmat, preferred_element_type=jnp.float32)
    tt = lax.broadcasted_iota(jnp.int32, (OUT, OUT), 0)
    oo = lax.broadcasted_iota(jnp.int32, (OUT, OUT), 1)
    pmat = (oo == (tt % H) * C + tt // H).astype(jnp.float32)
    agg = jnp.dot(a / dt, pmat, preferred_element_type=jnp.float32)
    scale = g_ref[...] * (1.0 / jnp.sqrt(1.0 + 1e-5))
    o = (agg + b_ref[...]) * scale + bb_ref[...]
    out_ref[...] = jnp.maximum(o, 0.0)


def kernel(x, edge_index, W, att_src, att_dst, bias, bn_weight, bn_bias):
    f32 = jnp.float32
    t = jnp.arange(OUT)
    Wp = W[:, (t % H) * C + (t // H)]
    atsf = att_src.T.reshape(1, OUT)
    atdf = att_dst.T.reshape(1, OUT)
    src = edge_index[0].astype(jnp.int32)
    dst = edge_index[1].astype(jnp.int32)

    ht, asrc, adst, mh = pl.pallas_call(
        _proj_body,
        out_shape=[
            jax.ShapeDtypeStruct((N, OUT), f32),
            jax.ShapeDtypeStruct((N, H), f32),
            jax.ShapeDtypeStruct((N, H), f32),
            jax.ShapeDtypeStruct((1, H), f32),
        ],
    )(x, Wp, atsf, atdf)

    sc_edge = pl.kernel(
        _sc_body,
        out_type=[
            jax.ShapeDtypeStruct((NC, NP, OUT), f32),
            jax.ShapeDtypeStruct((NC, NP, H), f32),
        ],
        mesh=plsc.VectorSubcoreMesh(core_axis_name="c", subcore_axis_name="s"),
        compiler_params=pltpu.CompilerParams(use_tc_tiling_on_sc=False),
        scratch_types=[
            pltpu.VMEM_SHARED((NP, OUT), f32),
            pltpu.VMEM_SHARED((NP, H), f32),
            pltpu.VMEM((K,), jnp.int32),
            pltpu.VMEM((K,), jnp.int32),
            pltpu.VMEM((NCHUNK, K), jnp.int32),
            pltpu.VMEM((K, OUT), f32),
            pltpu.VMEM((K, OUT), f32),
            pltpu.VMEM((K, H), f32),
            pltpu.VMEM((K, H), f32),
            pltpu.VMEM((K, H), f32),
            pltpu.VMEM((K, H), f32),
            pltpu.VMEM((K, OUT), f32),
            pltpu.VMEM((K, OUT), f32),
            pltpu.VMEM((K, H), f32),
            pltpu.VMEM((K, H), f32),
            pltpu.VMEM((H,), f32),
            pltpu.SemaphoreType.DMA,
            pltpu.SemaphoreType.DMA,
            pltpu.SemaphoreType.DMA,
            pltpu.SemaphoreType.DMA,
            pltpu.SemaphoreType.DMA,
            pltpu.SemaphoreType.DMA,
        ],
    )
    acc, den = sc_edge(src.reshape(NW * NCHUNK, K), dst.reshape(NW, NCHUNK, K),
                       ht, asrc, adst, mh.reshape(H),
                       jnp.zeros((NP, OUT), f32), jnp.zeros((NP, H), f32))

    out = pl.pallas_call(
        _epi_body,
        out_shape=jax.ShapeDtypeStruct((N, OUT), f32),
    )(acc, den, bias.reshape(1, OUT), bn_weight.reshape(1, OUT),
      bn_bias.reshape(1, OUT))
    return out
